# Initial kernel scaffold; baseline (speedup 1.0000x reference)
import jax
import jax.numpy as jnp
from jax.experimental import pallas as pl


def kernel(x, edge_index, edge_weight, W1, b1, W2, b2):
    raise NotImplementedError("write your pallas kernel here")


# speedup vs baseline: 19.3183x; 19.3183x over previous
import dataclasses
import functools

import jax
import jax.numpy as jnp
from jax import lax
from jax.experimental import pallas as pl
from jax.experimental.pallas import tpu as pltpu
from jax.experimental.pallas import tpu_sc as plsc

N = 100000
E = 1600000
DF = 128
DH = 16
DO = 2

NC = 2
NS = 16
NW = NC * NS

CHUNK = 128
ROWS_PER_W = 400
E_PAD = NW * ROWS_PER_W * CHUNK
R2D = E_PAD // CHUNK
BLK_ROWS = 8
BE = BLK_ROWS * CHUNK
N_ITERS = ROWS_PER_W // BLK_ROWS

EPT = E_PAD // NW
BDEG = 2048
N_PAD = 100096
RP = N_PAD // NS
ZROWS = 16

_mesh = plsc.VectorSubcoreMesh(core_axis_name="c", subcore_axis_name="s")

_sc_params = pltpu.CompilerParams()
if "needs_layout_passes" in pltpu.CompilerParams.__dataclass_fields__:
    _sc_params = dataclasses.replace(_sc_params, needs_layout_passes=False)
if "use_tc_tiling_on_sc" in pltpu.CompilerParams.__dataclass_fields__:
    _sc_params = dataclasses.replace(_sc_params, use_tc_tiling_on_sc=False)


@functools.partial(
    pl.kernel,
    out_type=jax.ShapeDtypeStruct((NW, 1, N), jnp.float32),
    mesh=_mesh,
    compiler_params=_sc_params,
    scratch_types=[
        pltpu.VMEM((N,), jnp.float32),
        pltpu.VMEM((BDEG,), jnp.int32),
        pltpu.VMEM((BDEG,), jnp.float32),
    ],
)
def _deg_kernel(dst_hbm, w_hbm, out_hbm, deg_v, dst_v, w_v):
    wid = lax.axis_index("s") * NC + lax.axis_index("c")

    @pl.loop(0, N, step=16)
    def _zero(i):
        deg_v[pl.ds(i, 16)] = jnp.zeros((16,), jnp.float32)

    base = wid * EPT

    @pl.loop(0, EPT, step=BDEG)
    def _block(off):
        pltpu.sync_copy(dst_hbm.at[pl.ds(base + off, BDEG)], dst_v)
        pltpu.sync_copy(w_hbm.at[pl.ds(base + off, BDEG)], w_v)

        @pl.loop(0, BDEG, step=16, unroll=4)
        def _scat(j):
            plsc.addupdate_scatter(
                deg_v, [dst_v[pl.ds(j, 16)]], w_v[pl.ds(j, 16)]
            )

    pltpu.sync_copy(deg_v, out_hbm.at[wid, 0])


@functools.partial(
    pl.kernel,
    out_type=jax.ShapeDtypeStruct((NC, N_PAD, DH), jnp.float32),
    mesh=_mesh,
    compiler_params=_sc_params,
    scratch_types=[
        pltpu.VMEM((BLK_ROWS, CHUNK), jnp.int32),
        pltpu.VMEM((BLK_ROWS, CHUNK), jnp.int32),
        pltpu.VMEM((BE,), jnp.float32),
        pltpu.VMEM((BE, DH), jnp.float32),
        pltpu.VMEM((ZROWS, DH), jnp.float32),
        pltpu.VMEM_SHARED((N_PAD, DH), jnp.float32),
        pltpu.SemaphoreType.DMA,
        pltpu.SemaphoreType.DMA,
    ],
)
def _agg_kernel(src2_hbm, dst2_hbm, w_hbm, y_hbm, out_hbm,
                src_v, dst_v, w_v, rows_v, zero_v, acc_sh, gsem, ssem):
    cid = lax.axis_index("c")
    sid = lax.axis_index("s")
    wid = sid * NC + cid

    @pl.loop(0, ZROWS)
    def _zrow(i):
        zero_v[i] = jnp.zeros((DH,), jnp.float32)

    @pl.loop(0, RP, step=ZROWS)
    def _zacc(r):
        pltpu.sync_copy(zero_v, acc_sh.at[pl.ds(sid * RP + r, ZROWS)])

    plsc.subcore_barrier()

    base_row = wid * ROWS_PER_W

    @pl.loop(0, N_ITERS)
    def _block(blk):
        row0 = base_row + blk * BLK_ROWS
        pltpu.sync_copy(src2_hbm.at[pl.ds(row0, BLK_ROWS)], src_v)
        pltpu.sync_copy(dst2_hbm.at[pl.ds(row0, BLK_ROWS)], dst_v)
        pltpu.sync_copy(w_hbm.at[pl.ds(row0 * CHUNK, BE)], w_v)

        gathers = []
        for j in range(BLK_ROWS):
            gathers.append(pltpu.async_copy(
                y_hbm.at[src_v.at[j]],
                rows_v.at[pl.ds(j * CHUNK, CHUNK)],
                gsem,
            ))
        for g in gathers:
            g.wait()

        @pl.loop(0, BE, step=8, unroll=1)
        def _scale(e0):
            for u in range(8):
                j = e0 + u
                wj = plsc.load_gather(w_v, [jnp.full((16,), j, jnp.int32)])
                rows_v[j] = rows_v[j] * wj

        scatters = []
        for j in range(BLK_ROWS):
            scatters.append(pltpu.async_copy(
                rows_v.at[pl.ds(j * CHUNK, CHUNK)],
                acc_sh.at[dst_v.at[j]],
                ssem,
                add=True,
            ))
        for s in scatters:
            s.wait()

    plsc.subcore_barrier()
    pltpu.sync_copy(acc_sh.at[pl.ds(sid * RP, RP)],
                    out_hbm.at[cid, pl.ds(sid * RP, RP)])


BLK = 1024
GRID = pl.cdiv(N, BLK)


def _mm1_body(x_ref, w_ref, o_ref):
    o_ref[...] = jax.lax.dot_general(
        x_ref[...], w_ref[...], (((1,), (0,)), ((), ())),
        preferred_element_type=jnp.float32)


def _mm1(x, W1):
    return pl.pallas_call(
        _mm1_body,
        grid=(GRID,),
        in_specs=[
            pl.BlockSpec((BLK, DF), lambda i: (i, 0)),
            pl.BlockSpec((DF, DH), lambda i: (0, 0)),
        ],
        out_specs=pl.BlockSpec((BLK, DH), lambda i: (i, 0)),
        out_shape=jax.ShapeDtypeStruct((N, DH), jnp.float32),
    )(x, W1)


def _prep_body(degp_ref, xw_ref, dis_ref, y1_ref):
    deg = 1.0 + jnp.sum(degp_ref[...], axis=0)
    dis = jax.lax.rsqrt(deg)
    dis_ref[...] = dis[:, None]
    y1_ref[...] = xw_ref[...] * dis[:, None]


def _prep(degp, xw):
    return pl.pallas_call(
        _prep_body,
        grid=(GRID,),
        in_specs=[
            pl.BlockSpec((NW, BLK), lambda i: (0, i)),
            pl.BlockSpec((BLK, DH), lambda i: (i, 0)),
        ],
        out_specs=[
            pl.BlockSpec((BLK, 1), lambda i: (i, 0)),
            pl.BlockSpec((BLK, DH), lambda i: (i, 0)),
        ],
        out_shape=[
            jax.ShapeDtypeStruct((N, 1), jnp.float32),
            jax.ShapeDtypeStruct((N, DH), jnp.float32),
        ],
    )(degp, xw)


def _comb1_body(accp_ref, xw_ref, dis_ref, b1_ref, h_ref, y2_ref):
    dis = dis_ref[...]
    acc = accp_ref[0] + accp_ref[1]
    pre = acc * dis + xw_ref[...] * (dis * dis) + b1_ref[...]
    h = jnp.maximum(pre, 0.0)
    h_ref[...] = h
    y2_ref[...] = h * dis


def _comb1(accp, xw, dis, b1):
    return pl.pallas_call(
        _comb1_body,
        grid=(GRID,),
        in_specs=[
            pl.BlockSpec((NC, BLK, DH), lambda i: (0, i, 0)),
            pl.BlockSpec((BLK, DH), lambda i: (i, 0)),
            pl.BlockSpec((BLK, 1), lambda i: (i, 0)),
            pl.BlockSpec((1, DH), lambda i: (0, 0)),
        ],
        out_specs=[
            pl.BlockSpec((BLK, DH), lambda i: (i, 0)),
            pl.BlockSpec((BLK, DH), lambda i: (i, 0)),
        ],
        out_shape=[
            jax.ShapeDtypeStruct((N, DH), jnp.float32),
            jax.ShapeDtypeStruct((N, DH), jnp.float32),
        ],
    )(accp, xw, dis, b1)


def _comb2_body(accp_ref, h_ref, dis_ref, w2_ref, b2_ref, o_ref):
    dis = dis_ref[...]
    acc = accp_ref[0] + accp_ref[1]
    g = acc * dis + h_ref[...] * (dis * dis)
    o_ref[...] = jax.lax.dot_general(
        g, w2_ref[...], (((1,), (0,)), ((), ())),
        preferred_element_type=jnp.float32) + b2_ref[...]


def _comb2(accp, h, dis, W2, b2):
    return pl.pallas_call(
        _comb2_body,
        grid=(GRID,),
        in_specs=[
            pl.BlockSpec((NC, BLK, DH), lambda i: (0, i, 0)),
            pl.BlockSpec((BLK, DH), lambda i: (i, 0)),
            pl.BlockSpec((BLK, 1), lambda i: (i, 0)),
            pl.BlockSpec((DH, DO), lambda i: (0, 0)),
            pl.BlockSpec((1, DO), lambda i: (0, 0)),
        ],
        out_specs=pl.BlockSpec((BLK, DO), lambda i: (i, 0)),
        out_shape=jax.ShapeDtypeStruct((N, DO), jnp.float32),
    )(accp, h, dis, W2, b2)


def kernel(x, edge_index, edge_weight, W1, b1, W2, b2):
    src = edge_index[0]
    dst = edge_index[1]
    pad = E_PAD - E
    src_p = jnp.pad(src, (0, pad))
    dst_p = jnp.pad(dst, (0, pad))
    w_p = jnp.pad(edge_weight, (0, pad))
    src2 = src_p.reshape(R2D, CHUNK)
    dst2 = dst_p.reshape(R2D, CHUNK)

    degp = _deg_kernel(dst_p, w_p).reshape(NW, N)
    xw = _mm1(x, W1)
    dis, y1 = _prep(degp, xw)

    acc1 = _agg_kernel(src2, dst2, w_p, y1)
    h, y2 = _comb1(acc1, xw, dis, b1.reshape(1, DH))

    acc2 = _agg_kernel(src2, dst2, w_p, y2)
    out = _comb2(acc2, h, dis, W2, b2.reshape(1, DO))
    return (h, out)
